# baseline (speedup 1.0000x reference)
import jax
import jax.numpy as jnp
from jax import lax
from jax.experimental import pallas as pl
from jax.experimental.pallas import tpu as pltpu
from jax.experimental.pallas import tpu_sc as plsc

N_SRC = 10000
N_DST = 10000
E = 320000
D = 128

NC = 2
NS = 16
NW = NC * NS
EPW = E // NW
K = 80
NCHUNK = EPW // K
ROWS_PT = 624
TAIL0 = NS * ROWS_PT
TAIL = N_DST - TAIL0


BLK_CH = 25
NBLK = NCHUNK // BLK_CH
BLK_E = BLK_CH * K


def _sc_body(x_hbm, src_hbm, dst_hbm, z128_hbm,
             agg_out, deg_out,
             src_blk_a, src_blk_b, dst_blk_a, dst_blk_b, dst2d,
             rows_a, rows_b, deg_l, agg_sh,
             gsem_a, gsem_b, bsem_a, bsem_b):
    c = lax.axis_index("c")
    s = lax.axis_index("s")
    wid = s * NC + c
    r0 = s * ROWS_PT
    rows = (rows_a, rows_b)
    gsem = (gsem_a, gsem_b)
    bsem = (bsem_a, bsem_b)
    src_blk = (src_blk_a, src_blk_b)
    dst_blk = (dst_blk_a, dst_blk_b)

    pltpu.sync_copy(z128_hbm.at[pl.ds(0, ROWS_PT)], agg_sh.at[pl.ds(r0, ROWS_PT)])

    @pl.when(s == NS - 1)
    def _():
        pltpu.sync_copy(z128_hbm.at[pl.ds(0, TAIL)], agg_sh.at[pl.ds(TAIL0, TAIL)])

    def zero(i, carry):
        deg_l[pl.ds(i * 16, 16)] = jnp.zeros((16,), jnp.float32)
        return carry
    lax.fori_loop(0, N_DST // 16, zero, 0)

    plsc.subcore_barrier()

    ones = jnp.ones((16,), jnp.float32)

    def fire_blk(b, sb):
        base = wid * EPW + b * BLK_E
        pltpu.async_copy(src_hbm.at[pl.ds(base, BLK_E)], src_blk[sb], bsem[sb])
        pltpu.async_copy(dst_hbm.at[pl.ds(base, BLK_E)], dst_blk[sb], bsem[sb])

    def drain_blk(sb):
        pltpu.make_async_copy(src_hbm.at[pl.ds(0, BLK_E)], src_blk[sb], bsem[sb]).wait()
        pltpu.make_async_copy(dst_hbm.at[pl.ds(0, BLK_E)], dst_blk[sb], bsem[sb]).wait()

    def fire_g(sb, local, p):
        pltpu.async_copy(x_hbm.at[src_blk[sb].at[pl.ds(local * K, K)]],
                         rows[p], gsem[p])

    def drain_g(sb, local, p):
        pltpu.make_async_copy(x_hbm.at[src_blk[sb].at[pl.ds(local * K, K)]],
                              rows[p], gsem[p]).wait()

    def consume(sb, local, p):
        def inner(j, c2):
            idx = dst_blk[sb][pl.ds(local * K + j * 16, 16)]
            dst2d[0, pl.ds(j * 16, 16)] = idx
            plsc.addupdate_scatter(deg_l, [idx], ones)
            return c2
        lax.fori_loop(0, K // 16, inner, 0)
        pltpu.sync_copy(rows[p], agg_sh.at[dst2d.at[0]], add=True)

    fire_blk(0, 0)
    drain_blk(0)
    fire_blk(1, 1)
    fire_g(0, 0, 0)

    for b in range(NBLK):
        sb = b % 2
        pb = b % 2

        def body(g, carry, b=b, sb=sb, pb=pb):
            for pp in (0, 1):
                local = 2 * g + pp
                p = (pb + pp) % 2
                drain_g(sb, local, p)
                fire_g(sb, local + 1, 1 - p)
                consume(sb, local, p)
            return carry
        lax.fori_loop(0, (BLK_CH - 1) // 2, body, 0)

        pl_last = (pb + BLK_CH - 1) % 2
        drain_g(sb, BLK_CH - 1, pl_last)
        if b + 1 < NBLK:
            drain_blk(1 - sb)
            fire_g(1 - sb, 0, 1 - pl_last)
        consume(sb, BLK_CH - 1, pl_last)
        if b + 2 < NBLK:
            fire_blk(b + 2, sb)

    plsc.subcore_barrier()

    pltpu.sync_copy(agg_sh.at[pl.ds(r0, ROWS_PT)], agg_out.at[c, pl.ds(r0, ROWS_PT)])

    @pl.when(s == NS - 1)
    def _():
        pltpu.sync_copy(agg_sh.at[pl.ds(TAIL0, TAIL)], agg_out.at[c, pl.ds(TAIL0, TAIL)])

    pltpu.sync_copy(deg_l, deg_out.at[pl.ds(wid * N_DST, N_DST)])


def _sc_aggregate(x_src, src, dst, z128):
    f = pl.kernel(
        _sc_body,
        out_type=[
            jax.ShapeDtypeStruct((NC, N_DST, D), jnp.float32),
            jax.ShapeDtypeStruct((NW * N_DST,), jnp.float32),
        ],
        mesh=plsc.VectorSubcoreMesh(core_axis_name="c", subcore_axis_name="s"),
        scratch_types=[
            pltpu.VMEM((BLK_E,), jnp.int32),
            pltpu.VMEM((BLK_E,), jnp.int32),
            pltpu.VMEM((BLK_E,), jnp.int32),
            pltpu.VMEM((BLK_E,), jnp.int32),
            pltpu.VMEM((1, K), jnp.int32),
            pltpu.VMEM((K, D), jnp.float32),
            pltpu.VMEM((K, D), jnp.float32),
            pltpu.VMEM((N_DST,), jnp.float32),
            pltpu.VMEM_SHARED((N_DST, D), jnp.float32),
        ] + [pltpu.SemaphoreType.DMA] * 4,
        compiler_params=pltpu.CompilerParams(needs_layout_passes=False),
    )
    return f(x_src, src, dst, z128)


def _tc_body(p_ref, d_ref, xd_ref, ws_ref, wd_ref, wm_ref,
             bs_ref, bd_ref, bm_ref, o_ref):
    a = p_ref[0] + p_ref[1]
    deg = jnp.sum(d_ref[...], axis=1, keepdims=True)
    inv = 1.0 / jnp.maximum(deg, 1.0)
    scale = jnp.minimum(deg, 1.0)
    t = jnp.dot(a, ws_ref[...], preferred_element_type=jnp.float32)
    agg_n = t * inv + bs_ref[...] * scale
    out = (jnp.dot(agg_n, wm_ref[...], preferred_element_type=jnp.float32)
           + jnp.dot(xd_ref[...], wd_ref[...], preferred_element_type=jnp.float32)
           + bm_ref[...] + bd_ref[...])
    o_ref[...] = jnp.maximum(out, 0.0)


def _tc_finish(parts, deg_nt, x_dst, ws_t, wd_t, wm_t, bs, bd, bm):
    B = 1000
    return pl.pallas_call(
        _tc_body,
        grid=(N_DST // B,),
        in_specs=[
            pl.BlockSpec((NC, B, D), lambda i: (0, i, 0)),
            pl.BlockSpec((B, NW), lambda i: (i, 0)),
            pl.BlockSpec((B, D), lambda i: (i, 0)),
            pl.BlockSpec((D, D), lambda i: (0, 0)),
            pl.BlockSpec((D, D), lambda i: (0, 0)),
            pl.BlockSpec((D, D), lambda i: (0, 0)),
            pl.BlockSpec((1, D), lambda i: (0, 0)),
            pl.BlockSpec((1, D), lambda i: (0, 0)),
            pl.BlockSpec((1, D), lambda i: (0, 0)),
        ],
        out_specs=pl.BlockSpec((B, D), lambda i: (i, 0)),
        out_shape=jax.ShapeDtypeStruct((N_DST, D), jnp.float32),
    )(parts, deg_nt, x_dst, ws_t, wd_t, wm_t, bs, bd, bm)


def kernel(x_src, x_dst, edge_index, W_src, b_src, W_dst, b_dst, W_m, b_m):
    src = edge_index[0]
    dst = edge_index[1]
    z128 = jnp.zeros((ROWS_PT, D), jnp.float32)
    parts, deg_flat = _sc_aggregate(x_src, src, dst, z128)
    deg_nt = deg_flat.reshape(NW, N_DST).T
    return _tc_finish(parts, deg_nt, x_dst, W_src.T, W_dst.T, W_m.T,
                      b_src[None, :], b_dst[None, :], b_m[None, :])


# speedup vs baseline: 1.1516x; 1.1516x over previous
import jax
import jax.numpy as jnp
from jax import lax
from jax.experimental import pallas as pl
from jax.experimental.pallas import tpu as pltpu
from jax.experimental.pallas import tpu_sc as plsc

N_SRC = 10000
N_DST = 10000
E = 320000
D = 128

NC = 2
NS = 16
NW = NC * NS
EPW = E // NW
K = 80
NCHUNK = EPW // K
ROWS_PT = 624
TAIL0 = NS * ROWS_PT
TAIL = N_DST - TAIL0


BLK_CH = 25
NBLK = NCHUNK // BLK_CH
BLK_E = BLK_CH * K


def _sc_body(x_hbm, src_hbm, dst_hbm, z128_hbm,
             agg_out, deg_out,
             src_blk, dst_blk, dst2d,
             rows_a, rows_b, rows_c, deg_l, agg_sh,
             gsem_a, gsem_b, gsem_c, bsem):
    c = lax.axis_index("c")
    s = lax.axis_index("s")
    wid = s * NC + c
    r0 = s * ROWS_PT
    rows = (rows_a, rows_b, rows_c)
    gsem = (gsem_a, gsem_b, gsem_c)

    pltpu.sync_copy(z128_hbm.at[pl.ds(0, ROWS_PT)], agg_sh.at[pl.ds(r0, ROWS_PT)])

    @pl.when(s == NS - 1)
    def _():
        pltpu.sync_copy(z128_hbm.at[pl.ds(0, TAIL)], agg_sh.at[pl.ds(TAIL0, TAIL)])

    def zero(i, carry):
        deg_l[pl.ds(i * 16, 16)] = jnp.zeros((16,), jnp.float32)
        return carry
    lax.fori_loop(0, N_DST // 16, zero, 0)

    plsc.subcore_barrier()

    ones = jnp.ones((16,), jnp.float32)

    def fire_blk(b):
        base = wid * EPW + b * BLK_E
        pltpu.async_copy(src_hbm.at[pl.ds(base, BLK_E)], src_blk, bsem)
        pltpu.async_copy(dst_hbm.at[pl.ds(base, BLK_E)], dst_blk, bsem)

    def drain_blk():
        pltpu.make_async_copy(src_hbm.at[pl.ds(0, BLK_E)], src_blk, bsem).wait()
        pltpu.make_async_copy(dst_hbm.at[pl.ds(0, BLK_E)], dst_blk, bsem).wait()

    def fire_g(local, q):
        pltpu.async_copy(x_hbm.at[src_blk.at[pl.ds(local * K, K)]],
                         rows[q], gsem[q])

    def drain_g(local, q):
        pltpu.make_async_copy(x_hbm.at[src_blk.at[pl.ds(local * K, K)]],
                              rows[q], gsem[q]).wait()

    def consume(local, q):
        def inner(j, c2):
            idx = dst_blk[pl.ds(local * K + j * 16, 16)]
            dst2d[0, pl.ds(j * 16, 16)] = idx
            plsc.addupdate_scatter(deg_l, [idx], ones)
            return c2
        lax.fori_loop(0, K // 16, inner, 0)
        pltpu.sync_copy(rows[q], agg_sh.at[dst2d.at[0]], add=True)

    fire_blk(0)
    drain_blk()
    fire_g(0, 0)
    fire_g(1, 1)

    for b in range(NBLK):
        def body(g, carry, b=b):
            for pp in (0, 1, 2):
                local = 3 * g + pp
                q = (b + pp) % 3
                drain_g(local, q)
                consume(local, q)

                @pl.when(local + 2 < BLK_CH)
                def _():
                    fire_g(local + 2, (q + 2) % 3)
            return carry
        lax.fori_loop(0, (BLK_CH - 1) // 3, body, 0)

        q_last = (b + BLK_CH - 1) % 3
        drain_g(BLK_CH - 1, q_last)
        consume(BLK_CH - 1, q_last)
        if b + 1 < NBLK:
            fire_blk(b + 1)
            drain_blk()
            fire_g(0, (b + 1) % 3)
            fire_g(1, (b + 2) % 3)

    plsc.subcore_barrier()

    pltpu.sync_copy(agg_sh.at[pl.ds(r0, ROWS_PT)], agg_out.at[c, pl.ds(r0, ROWS_PT)])

    @pl.when(s == NS - 1)
    def _():
        pltpu.sync_copy(agg_sh.at[pl.ds(TAIL0, TAIL)], agg_out.at[c, pl.ds(TAIL0, TAIL)])

    pltpu.sync_copy(deg_l, deg_out.at[pl.ds(wid * N_DST, N_DST)])


def _sc_aggregate(x_src, src, dst, z128):
    f = pl.kernel(
        _sc_body,
        out_type=[
            jax.ShapeDtypeStruct((NC, N_DST, D), jnp.float32),
            jax.ShapeDtypeStruct((NW * N_DST,), jnp.float32),
        ],
        mesh=plsc.VectorSubcoreMesh(core_axis_name="c", subcore_axis_name="s"),
        scratch_types=[
            pltpu.VMEM((BLK_E,), jnp.int32),
            pltpu.VMEM((BLK_E,), jnp.int32),
            pltpu.VMEM((1, K), jnp.int32),
            pltpu.VMEM((K, D), jnp.float32),
            pltpu.VMEM((K, D), jnp.float32),
            pltpu.VMEM((K, D), jnp.float32),
            pltpu.VMEM((N_DST,), jnp.float32),
            pltpu.VMEM_SHARED((N_DST, D), jnp.float32),
        ] + [pltpu.SemaphoreType.DMA] * 4,
        compiler_params=pltpu.CompilerParams(needs_layout_passes=False),
    )
    return f(x_src, src, dst, z128)


def _tc_body(p_ref, d_ref, xd_ref, ws_ref, wd_ref, wm_ref,
             bs_ref, bd_ref, bm_ref, o_ref):
    a = p_ref[0] + p_ref[1]
    deg = jnp.sum(d_ref[...], axis=1, keepdims=True)
    inv = 1.0 / jnp.maximum(deg, 1.0)
    scale = jnp.minimum(deg, 1.0)
    t = jnp.dot(a, ws_ref[...], preferred_element_type=jnp.float32)
    agg_n = t * inv + bs_ref[...] * scale
    out = (jnp.dot(agg_n, wm_ref[...], preferred_element_type=jnp.float32)
           + jnp.dot(xd_ref[...], wd_ref[...], preferred_element_type=jnp.float32)
           + bm_ref[...] + bd_ref[...])
    o_ref[...] = jnp.maximum(out, 0.0)


def _tc_finish(parts, deg_nt, x_dst, ws_t, wd_t, wm_t, bs, bd, bm):
    B = 1000
    return pl.pallas_call(
        _tc_body,
        grid=(N_DST // B,),
        in_specs=[
            pl.BlockSpec((NC, B, D), lambda i: (0, i, 0)),
            pl.BlockSpec((B, NW), lambda i: (i, 0)),
            pl.BlockSpec((B, D), lambda i: (i, 0)),
            pl.BlockSpec((D, D), lambda i: (0, 0)),
            pl.BlockSpec((D, D), lambda i: (0, 0)),
            pl.BlockSpec((D, D), lambda i: (0, 0)),
            pl.BlockSpec((1, D), lambda i: (0, 0)),
            pl.BlockSpec((1, D), lambda i: (0, 0)),
            pl.BlockSpec((1, D), lambda i: (0, 0)),
        ],
        out_specs=pl.BlockSpec((B, D), lambda i: (i, 0)),
        out_shape=jax.ShapeDtypeStruct((N_DST, D), jnp.float32),
    )(parts, deg_nt, x_dst, ws_t, wd_t, wm_t, bs, bd, bm)


def kernel(x_src, x_dst, edge_index, W_src, b_src, W_dst, b_dst, W_m, b_m):
    src = edge_index[0]
    dst = edge_index[1]
    z128 = jnp.zeros((ROWS_PT, D), jnp.float32)
    parts, deg_flat = _sc_aggregate(x_src, src, dst, z128)
    deg_nt = deg_flat.reshape(NW, N_DST).T
    return _tc_finish(parts, deg_nt, x_dst, W_src.T, W_dst.T, W_m.T,
                      b_src[None, :], b_dst[None, :], b_m[None, :])
